# baseline (speedup 1.0000x reference)
import functools

import jax
import jax.numpy as jnp
import numpy as np
from jax import lax
from jax.experimental import pallas as pl
from jax.experimental.pallas import tpu as pltpu

PACK = 4
DOT_DT = jnp.bfloat16


def _conv_packed(src_ref, wc_ref, we_ref, i, b_row, *, W4, TOP, M):
    cq = src_ref.shape[1] // 4
    wins = jnp.concatenate(
        [src_ref[TOP - W4:TOP - W4 + M, :],
         src_ref[TOP:TOP + M, :],
         src_ref[TOP + W4:TOP + W4 + M, :]], axis=1)
    accC = jnp.dot(wins, wc_ref[i], preferred_element_type=jnp.float32)
    ewins = jnp.concatenate(
        [wins[:, 0:2 * cq], wins[:, 4 * cq:6 * cq], wins[:, 8 * cq:10 * cq]],
        axis=1)
    accE = jnp.dot(ewins, we_ref[i], preferred_element_type=jnp.float32)
    cp = accC.shape[1]
    zrow = jnp.zeros((1, cp), jnp.float32)
    shL = jnp.concatenate([zrow, accE[:-1, :]], axis=0)
    shR = jnp.concatenate([accE[1:, :], zrow], axis=0)
    col = lax.broadcasted_iota(jnp.int32, (M, 1), 0) % W4
    lane = lax.broadcasted_iota(jnp.int32, (M, cp), 1)
    out = accC + jnp.where((lane < cq) & (col != 0), shL, 0.0)
    out = out + jnp.where((lane >= cq) & (col != W4 - 1), shR, 0.0)
    return out + b_row


def _group_kernel(x_ref, wc_ref, we_ref, ball_ref,
                  wd1s_ref, bd1s_ref, wd2s_ref, bd2s_ref,
                  o_ref, h_ref, hb_ref, rp_ref,
                  *, W4, TOP, M, n_blocks):
    h_ref[...] = x_ref[...]
    hb_ref[...] = x_ref[...].astype(DOT_DT)
    rp_ref[...] = jnp.zeros_like(rp_ref)

    for i in range(n_blocks):
        r1 = _conv_packed(hb_ref, wc_ref, we_ref, i, ball_ref[i],
                          W4=W4, TOP=TOP, M=M)
        rp_ref[TOP:TOP + M, :] = jnp.maximum(r1, 0.0).astype(DOT_DT)
        r = _conv_packed(rp_ref, wc_ref, we_ref, n_blocks + i,
                         ball_ref[n_blocks + i], W4=W4, TOP=TOP, M=M)
        y = jnp.mean(r, axis=0, keepdims=True)
        z = jnp.dot(y, wd1s_ref[i], preferred_element_type=jnp.float32)
        z = jnp.maximum(z + bd1s_ref[i], 0.0)
        s = jnp.dot(z, wd2s_ref[i], preferred_element_type=jnp.float32)
        s = jax.nn.sigmoid(s + bd2s_ref[i])
        hn = h_ref[TOP:TOP + M, :] + r * s
        h_ref[TOP:TOP + M, :] = hn
        hb_ref[TOP:TOP + M, :] = hn.astype(DOT_DT)

    conv = _conv_packed(hb_ref, wc_ref, we_ref, 2 * n_blocks,
                        ball_ref[2 * n_blocks], W4=W4, TOP=TOP, M=M)
    o_ref[...] = conv + x_ref[TOP:TOP + M, :]


_SLOT_POS = np.array([0, 3, 1, 2])
_POS_SLOT = np.argsort(_SLOT_POS)
_D = np.array([-1, 0, 1])
_TOEPLITZ_IDX = (4 * _D[:, None, None] + _SLOT_POS[None, :, None]
                 - _SLOT_POS[None, None, :] + 7)


def _pack_conv_w(ws, C):
    B = ws.shape[0]
    P = PACK
    wpad = jnp.pad(ws, ((0, 0), (0, 0), (6, 6), (0, 0), (0, 0)))
    wp = wpad[:, :, _TOEPLITZ_IDX]
    wp = jnp.transpose(wp, (0, 2, 1, 3, 5, 4, 6))
    wc = wp[:, 1].reshape(B, 3 * P * C, P * C)
    we = (wp[:, 0] + wp[:, 2])[:, :, :2]
    we = we.reshape(B, 3 * 2 * C, P * C)
    return wc, we


def _bcast_spec(shape):
    return pl.BlockSpec(shape, lambda n: (0,) * len(shape))


def kernel(x, b0_w1, b0_b1, b0_w2, b0_b2, b0_wd1, b0_bd1, b0_wd2, b0_bd2, b1_w1, b1_b1, b1_w2, b1_b2, b1_wd1, b1_bd1, b1_wd2, b1_bd2, b2_w1, b2_b1, b2_w2, b2_b2, b2_wd1, b2_bd1, b2_wd2, b2_bd2, b3_w1, b3_b1, b3_w2, b3_b2, b3_wd1, b3_bd1, b3_wd2, b3_bd2, b4_w1, b4_b1, b4_w2, b4_b2, b4_wd1, b4_bd1, b4_wd2, b4_bd2, b5_w1, b5_b1, b5_w2, b5_b2, b5_wd1, b5_bd1, b5_wd2, b5_bd2, b6_w1, b6_b1, b6_w2, b6_b2, b6_wd1, b6_bd1, b6_wd2, b6_bd2, b7_w1, b7_b1, b7_w2, b7_b2, b7_wd1, b7_bd1, b7_wd2, b7_bd2, wf, bf):
    blocks = [
        dict(w1=b0_w1, b1=b0_b1, w2=b0_w2, b2=b0_b2, wd1=b0_wd1, bd1=b0_bd1, wd2=b0_wd2, bd2=b0_bd2),
        dict(w1=b1_w1, b1=b1_b1, w2=b1_w2, b2=b1_b2, wd1=b1_wd1, bd1=b1_bd1, wd2=b1_wd2, bd2=b1_bd2),
        dict(w1=b2_w1, b1=b2_b1, w2=b2_w2, b2=b2_b2, wd1=b2_wd1, bd1=b2_bd1, wd2=b2_wd2, bd2=b2_bd2),
        dict(w1=b3_w1, b1=b3_b1, w2=b3_w2, b2=b3_b2, wd1=b3_wd1, bd1=b3_bd1, wd2=b3_wd2, bd2=b3_bd2),
        dict(w1=b4_w1, b1=b4_b1, w2=b4_w2, b2=b4_b2, wd1=b4_wd1, bd1=b4_bd1, wd2=b4_wd2, bd2=b4_bd2),
        dict(w1=b5_w1, b1=b5_b1, w2=b5_w2, b2=b5_b2, wd1=b5_wd1, bd1=b5_bd1, wd2=b5_wd2, bd2=b5_bd2),
        dict(w1=b6_w1, b1=b6_b1, w2=b6_w2, b2=b6_b2, wd1=b6_wd1, bd1=b6_bd1, wd2=b6_wd2, bd2=b6_bd2),
        dict(w1=b7_w1, b1=b7_b1, w2=b7_w2, b2=b7_b2, wd1=b7_wd1, bd1=b7_bd1, wd2=b7_wd2, bd2=b7_bd2),
    ]
    N, C, H, W = x.shape
    P = PACK
    W4 = W // P
    M = H * W4
    Cp = P * C
    Cr = blocks[0]["wd1"].shape[1]
    TOP = 32
    Lpad = TOP + M + TOP
    nb = len(blocks)

    w_raw = jnp.concatenate([jnp.stack([b["w1"] for b in blocks]),
                             jnp.stack([b["w2"] for b in blocks]),
                             wf[None]], axis=0).astype(DOT_DT)
    wallc, walle = _pack_conv_w(w_raw, C)
    b_raw = jnp.concatenate([jnp.stack([b["b1"] for b in blocks]),
                             jnp.stack([b["b2"] for b in blocks]),
                             bf[None]], axis=0)
    ball = jnp.tile(b_raw, (1, P)).reshape(2 * nb + 1, 1, Cp)
    CrP = 128
    wd1s = jnp.pad(jnp.tile(jnp.stack([b["wd1"] for b in blocks]) / P,
                            (1, P, 1)), ((0, 0), (0, 0), (0, CrP - Cr)))
    bd1s = jnp.pad(jnp.stack([b["bd1"] for b in blocks]),
                   ((0, 0), (0, CrP - Cr))).reshape(nb, 1, CrP)
    wd2s = jnp.pad(jnp.tile(jnp.stack([b["wd2"] for b in blocks]), (1, 1, P)),
                   ((0, 0), (0, CrP - Cr), (0, 0)))
    bd2s = jnp.tile(jnp.stack([b["bd2"] for b in blocks]),
                    (1, P)).reshape(nb, 1, Cp)

    x_flat = jnp.transpose(x, (0, 2, 3, 1)).reshape(N, H, W4, P, C)
    x_flat = x_flat[:, :, :, _SLOT_POS].reshape(N, M, Cp)
    xp = jnp.pad(x_flat, ((0, 0), (TOP, TOP), (0, 0)))

    body = functools.partial(_group_kernel, W4=W4, TOP=TOP, M=M, n_blocks=nb)
    out = pl.pallas_call(
        body,
        out_shape=jax.ShapeDtypeStruct((N, M, Cp), x.dtype),
        grid=(N,),
        in_specs=[
            pl.BlockSpec((pl.Squeezed(), Lpad, Cp), lambda n: (n, 0, 0)),
            _bcast_spec((2 * nb + 1, 3 * Cp, Cp)),
            _bcast_spec((2 * nb + 1, 6 * C, Cp)),
            _bcast_spec((2 * nb + 1, 1, Cp)),
            _bcast_spec((nb, Cp, CrP)), _bcast_spec((nb, 1, CrP)),
            _bcast_spec((nb, CrP, Cp)), _bcast_spec((nb, 1, Cp)),
        ],
        out_specs=pl.BlockSpec((pl.Squeezed(), M, Cp), lambda n: (n, 0, 0)),
        scratch_shapes=[pltpu.VMEM((Lpad, Cp), jnp.float32),
                        pltpu.VMEM((Lpad, Cp), DOT_DT),
                        pltpu.VMEM((Lpad, Cp), DOT_DT)],
        compiler_params=pltpu.CompilerParams(dimension_semantics=("parallel",)),
    )(xp, wallc, walle, ball, wd1s, bd1s, wd2s, bd2s)

    out = out.reshape(N, H, W4, P, C)[:, :, :, _POS_SLOT].reshape(N, H, W, C)
    return jnp.transpose(out, (0, 3, 1, 2))


# speedup vs baseline: 1.2651x; 1.2651x over previous
import functools

import jax
import jax.numpy as jnp
import numpy as np
from jax import lax
from jax.experimental import pallas as pl
from jax.experimental.pallas import tpu as pltpu

PACK = 4
DOT_DT = jnp.bfloat16


def _conv_packed(src_ref, wt_ref, i, b_row, *, W4, TOP, M):
    wins = jnp.concatenate(
        [src_ref[TOP - W4:TOP - W4 + M, :],
         src_ref[TOP:TOP + M, :],
         src_ref[TOP + W4:TOP + W4 + M, :]], axis=1)
    accC = jnp.dot(wins, wt_ref[i, 0], preferred_element_type=jnp.float32)
    accE = jnp.dot(wins, wt_ref[i, 1], preferred_element_type=jnp.float32)
    cp = accC.shape[1]
    zrow = jnp.zeros((1, cp), jnp.float32)
    shL = jnp.concatenate([zrow, accE[:-1, :]], axis=0)
    shR = jnp.concatenate([accE[1:, :], zrow], axis=0)
    col = lax.broadcasted_iota(jnp.int32, (M, 1), 0) % W4
    lane = lax.broadcasted_iota(jnp.int32, (M, cp), 1)
    out = accC + jnp.where((lane < cp // 2) & (col != 0), shL, 0.0)
    out = out + jnp.where((lane >= cp // 2) & (col != W4 - 1), shR, 0.0)
    return out + b_row


def _group_kernel(x_ref, wall_ref, ball_ref,
                  wd1s_ref, bd1s_ref, wd2s_ref, bd2s_ref,
                  o_ref, h_ref, hb_ref, rp_ref,
                  *, W4, TOP, M, n_blocks):
    h_ref[...] = x_ref[...]
    hb_ref[...] = x_ref[...].astype(DOT_DT)
    rp_ref[...] = jnp.zeros_like(rp_ref)

    for i in range(n_blocks):
        r1 = _conv_packed(hb_ref, wall_ref, i, ball_ref[i],
                          W4=W4, TOP=TOP, M=M)
        rp_ref[TOP:TOP + M, :] = jnp.maximum(r1, 0.0).astype(DOT_DT)
        r = _conv_packed(rp_ref, wall_ref, n_blocks + i, ball_ref[n_blocks + i],
                         W4=W4, TOP=TOP, M=M)
        y = jnp.mean(r, axis=0, keepdims=True)
        z = jnp.dot(y, wd1s_ref[i], preferred_element_type=jnp.float32)
        z = jnp.maximum(z + bd1s_ref[i], 0.0)
        s = jnp.dot(z, wd2s_ref[i], preferred_element_type=jnp.float32)
        s = jax.nn.sigmoid(s + bd2s_ref[i])
        hn = h_ref[TOP:TOP + M, :] + r * s
        h_ref[TOP:TOP + M, :] = hn
        hb_ref[TOP:TOP + M, :] = hn.astype(DOT_DT)

    conv = _conv_packed(hb_ref, wall_ref, 2 * n_blocks, ball_ref[2 * n_blocks],
                        W4=W4, TOP=TOP, M=M)
    o_ref[...] = conv + x_ref[TOP:TOP + M, :]


_D = np.array([-1, 0, 1])
_PI = np.arange(PACK)
_PO = np.arange(PACK)
_TOEPLITZ_IDX = (4 * _D[:, None, None] + _PI[None, :, None]
                 - _PO[None, None, :] + 7)


def _pack_conv_w(ws, C):
    B = ws.shape[0]
    P = PACK
    wpad = jnp.pad(ws, ((0, 0), (0, 0), (6, 6), (0, 0), (0, 0)))
    wp = wpad[:, :, _TOEPLITZ_IDX]
    wp = jnp.transpose(wp, (0, 2, 1, 3, 5, 4, 6))
    wp = jnp.stack([wp[:, 1], wp[:, 0] + wp[:, 2]], axis=1)
    return wp.reshape(B, 2, 3 * P * C, P * C)


def _bcast_spec(shape):
    return pl.BlockSpec(shape, lambda n: (0,) * len(shape))


def kernel(x, b0_w1, b0_b1, b0_w2, b0_b2, b0_wd1, b0_bd1, b0_wd2, b0_bd2, b1_w1, b1_b1, b1_w2, b1_b2, b1_wd1, b1_bd1, b1_wd2, b1_bd2, b2_w1, b2_b1, b2_w2, b2_b2, b2_wd1, b2_bd1, b2_wd2, b2_bd2, b3_w1, b3_b1, b3_w2, b3_b2, b3_wd1, b3_bd1, b3_wd2, b3_bd2, b4_w1, b4_b1, b4_w2, b4_b2, b4_wd1, b4_bd1, b4_wd2, b4_bd2, b5_w1, b5_b1, b5_w2, b5_b2, b5_wd1, b5_bd1, b5_wd2, b5_bd2, b6_w1, b6_b1, b6_w2, b6_b2, b6_wd1, b6_bd1, b6_wd2, b6_bd2, b7_w1, b7_b1, b7_w2, b7_b2, b7_wd1, b7_bd1, b7_wd2, b7_bd2, wf, bf):
    blocks = [
        dict(w1=b0_w1, b1=b0_b1, w2=b0_w2, b2=b0_b2, wd1=b0_wd1, bd1=b0_bd1, wd2=b0_wd2, bd2=b0_bd2),
        dict(w1=b1_w1, b1=b1_b1, w2=b1_w2, b2=b1_b2, wd1=b1_wd1, bd1=b1_bd1, wd2=b1_wd2, bd2=b1_bd2),
        dict(w1=b2_w1, b1=b2_b1, w2=b2_w2, b2=b2_b2, wd1=b2_wd1, bd1=b2_bd1, wd2=b2_wd2, bd2=b2_bd2),
        dict(w1=b3_w1, b1=b3_b1, w2=b3_w2, b2=b3_b2, wd1=b3_wd1, bd1=b3_bd1, wd2=b3_wd2, bd2=b3_bd2),
        dict(w1=b4_w1, b1=b4_b1, w2=b4_w2, b2=b4_b2, wd1=b4_wd1, bd1=b4_bd1, wd2=b4_wd2, bd2=b4_bd2),
        dict(w1=b5_w1, b1=b5_b1, w2=b5_w2, b2=b5_b2, wd1=b5_wd1, bd1=b5_bd1, wd2=b5_wd2, bd2=b5_bd2),
        dict(w1=b6_w1, b1=b6_b1, w2=b6_w2, b2=b6_b2, wd1=b6_wd1, bd1=b6_bd1, wd2=b6_wd2, bd2=b6_bd2),
        dict(w1=b7_w1, b1=b7_b1, w2=b7_w2, b2=b7_b2, wd1=b7_wd1, bd1=b7_bd1, wd2=b7_wd2, bd2=b7_bd2),
    ]
    N, C, H, W = x.shape
    P = PACK
    W4 = W // P
    M = H * W4
    Cp = P * C
    Cr = blocks[0]["wd1"].shape[1]
    TOP = 32
    Lpad = TOP + M + TOP
    nb = len(blocks)

    w_raw = jnp.concatenate([jnp.stack([b["w1"] for b in blocks]),
                             jnp.stack([b["w2"] for b in blocks]),
                             wf[None]], axis=0).astype(DOT_DT)
    wall = _pack_conv_w(w_raw, C)
    b_raw = jnp.concatenate([jnp.stack([b["b1"] for b in blocks]),
                             jnp.stack([b["b2"] for b in blocks]),
                             bf[None]], axis=0)
    ball = jnp.tile(b_raw, (1, P)).reshape(2 * nb + 1, 1, Cp)
    CrP = 128
    wd1s = jnp.pad(jnp.tile(jnp.stack([b["wd1"] for b in blocks]) / P,
                            (1, P, 1)), ((0, 0), (0, 0), (0, CrP - Cr)))
    bd1s = jnp.pad(jnp.stack([b["bd1"] for b in blocks]),
                   ((0, 0), (0, CrP - Cr))).reshape(nb, 1, CrP)
    wd2s = jnp.pad(jnp.tile(jnp.stack([b["wd2"] for b in blocks]), (1, 1, P)),
                   ((0, 0), (0, CrP - Cr), (0, 0)))
    bd2s = jnp.tile(jnp.stack([b["bd2"] for b in blocks]),
                    (1, P)).reshape(nb, 1, Cp)

    x_flat = jnp.transpose(x, (0, 2, 3, 1)).reshape(N, H, W4, P * C)
    xp = jnp.pad(x_flat.reshape(N, M, Cp), ((0, 0), (TOP, TOP), (0, 0)))

    body = functools.partial(_group_kernel, W4=W4, TOP=TOP, M=M, n_blocks=nb)
    out = pl.pallas_call(
        body,
        out_shape=jax.ShapeDtypeStruct((N, M, Cp), x.dtype),
        grid=(N,),
        in_specs=[
            pl.BlockSpec((pl.Squeezed(), Lpad, Cp), lambda n: (n, 0, 0)),
            _bcast_spec((2 * nb + 1, 2, 3 * Cp, Cp)),
            _bcast_spec((2 * nb + 1, 1, Cp)),
            _bcast_spec((nb, Cp, CrP)), _bcast_spec((nb, 1, CrP)),
            _bcast_spec((nb, CrP, Cp)), _bcast_spec((nb, 1, Cp)),
        ],
        out_specs=pl.BlockSpec((pl.Squeezed(), M, Cp), lambda n: (n, 0, 0)),
        scratch_shapes=[pltpu.VMEM((Lpad, Cp), jnp.float32),
                        pltpu.VMEM((Lpad, Cp), DOT_DT),
                        pltpu.VMEM((Lpad, Cp), DOT_DT)],
        compiler_params=pltpu.CompilerParams(dimension_semantics=("parallel",)),
    )(xp, wall, ball, wd1s, bd1s, wd2s, bd2s)

    out = out.reshape(N, H, W, C)
    return jnp.transpose(out, (0, 3, 1, 2))
